# baseline (speedup 1.0000x reference)
import functools

import jax
import jax.numpy as jnp
import numpy as np
from jax import lax
from jax.experimental import pallas as pl
from jax.experimental.pallas import tpu as pltpu
from jax.experimental.pallas import tpu_sc as plsc

N = 10000
E = 320000
HID = 128
NG = 50
NF = 128
NCOMP = 256
CUTOFF = 10.0
SHIFT = float(np.log(2.0))
LOG2E = float(np.log2(np.e))
LN2 = float(np.log(2.0))

NHALF = 2
E_SLICE = (128000, 192000)
E_OFF = (0, 128000)
NC = 2
NS = 16
NW = NC * NS
CHUNK = 40
NBUF = 3
N_PAD = 10240
ROWS_PT = N_PAD // NS
ZR = 40
BE = 3200
BN = 2048


def _wf_body(eat_ref, ew_ref, w0t_ref, b0_ref, w2t_ref, b2_ref, wf_ref):
    t = lax.dot_general(eat_ref[...], w0t_ref[...], (((0,), (0,)), ((), ())),
                        preferred_element_type=jnp.float32)
    t = t + b0_ref[...]
    t = jnp.log2(1.0 + jnp.exp2(t * LOG2E)) * LN2 - SHIFT
    wf = jnp.dot(t, w2t_ref[...], preferred_element_type=jnp.float32) + b2_ref[...]
    ew = ew_ref[0].reshape(BE, 1)
    c = CUTOFF / (1e-10 + ew * ew) - 1.0
    wf_ref[...] = wf * c


def _h_body(x_ref, w_ref, h_ref):
    h_ref[...] = jnp.dot(x_ref[...], w_ref[...], preferred_element_type=jnp.float32)


def _sc_body(half, h_hbm, wf_hbm, src_hbm, dst_hbm, out_hbm,
             srcall_v, dst_v, rows_v, wf_v, agg_sh,
             ssrc, sd, sg, sw):
    edges_pt = E_SLICE[half] // NW
    nchunk = edges_pt // CHUNK
    c = lax.axis_index("c")
    s = lax.axis_index("s")
    wid = s * NC + c
    base = E_OFF[half] + wid * edges_pt

    pltpu.async_copy(src_hbm.at[pl.ds(base, edges_pt)],
                     srcall_v.at[pl.ds(0, edges_pt)], ssrc)

    def zrow(r, cy):
        for j in range(NF // 16):
            rows_v[0][r, pl.ds(16 * j, 16)] = jnp.zeros((16,), jnp.float32)
        return cy

    lax.fori_loop(0, ZR, zrow, 0)
    for k in range(ROWS_PT // ZR):
        pltpu.sync_copy(rows_v[0], agg_sh.at[pl.ds(s * ROWS_PT + k * ZR, ZR)])
    plsc.subcore_barrier()
    pltpu.make_async_copy(src_hbm.at[pl.ds(base, edges_pt)],
                           srcall_v.at[pl.ds(0, edges_pt)], ssrc).wait()

    def issue(ci, b):
        off = base + ci * CHUNK
        pltpu.async_copy(dst_hbm.at[pl.ds(off, CHUNK)], dst_v[b], sd[b])
        pltpu.async_copy(h_hbm.at[srcall_v.at[pl.ds(ci * CHUNK, CHUNK)]],
                         rows_v[b], sg[b])
        pltpu.async_copy(wf_hbm.at[pl.ds(off - E_OFF[half], CHUNK)], wf_v[b], sw[b])

    def wait_in(ci, b):
        off = base + ci * CHUNK
        pltpu.make_async_copy(h_hbm.at[srcall_v.at[pl.ds(ci * CHUNK, CHUNK)]],
                              rows_v[b], sg[b]).wait()
        pltpu.make_async_copy(wf_hbm.at[pl.ds(off - E_OFF[half], CHUNK)],
                              wf_v[b], sw[b]).wait()
        pltpu.make_async_copy(dst_hbm.at[pl.ds(off, CHUNK)], dst_v[b], sd[b]).wait()

    def process(b):
        @plsc.parallel_loop(0, CHUNK, step=1, unroll=4)
        def _(r):
            for j in range(NF // 16):
                rows_v[b][r, pl.ds(16 * j, 16)] = (
                    rows_v[b][r, pl.ds(16 * j, 16)] * wf_v[b][r, pl.ds(16 * j, 16)])

        pltpu.sync_copy(rows_v[b], agg_sh.at[dst_v[b]], add=True)

    issue(0, 0)
    issue(1, 1)

    def step(i, carry):
        for par in range(NBUF):
            @pl.when(lax.rem(i, NBUF) == par)
            def _(par=par):
                @pl.when(i + 2 < nchunk)
                def _():
                    issue(i + 2, (par + 2) % NBUF)

                wait_in(i, par)
                process(par)

        return carry

    lax.fori_loop(0, nchunk, step, 0)
    plsc.subcore_barrier()
    for k in range(ROWS_PT // ZR):
        pltpu.sync_copy(agg_sh.at[pl.ds(s * ROWS_PT + k * ZR, ZR)],
                        out_hbm.at[c, pl.ds(s * ROWS_PT + k * ZR, ZR)])


def _final_body(p0_ref, p1_ref, idxc_ref, w2t_ref, b2_ref, wt_ref, b_ref, out_ref,
                acc_ref, cnt_ref):
    i = pl.program_id(0)
    agg = (p0_ref[0] + p0_ref[1]) + (p1_ref[0] + p1_ref[1])
    iota_c = lax.broadcasted_iota(jnp.int32, (NCOMP, BN), 0)
    sel = (idxc_ref[0] == iota_c).astype(jnp.float32)
    acc = jnp.dot(sel, agg, preferred_element_type=jnp.float32)
    cnt = jnp.sum(sel, axis=1, keepdims=True)

    @pl.when(i == 0)
    def _():
        acc_ref[...] = jnp.zeros_like(acc_ref)
        cnt_ref[...] = jnp.zeros_like(cnt_ref)

    acc_ref[...] += acc
    cnt_ref[...] += cnt

    @pl.when(i == pl.num_programs(0) - 1)
    def _():
        mean = acc_ref[...] / jnp.maximum(cnt_ref[...], 1.0)
        t = jnp.dot(mean, w2t_ref[...], preferred_element_type=jnp.float32)
        t = jax.nn.softplus(t + b2_ref[...]) - SHIFT
        out_ref[...] = (jnp.dot(t, wt_ref[...], preferred_element_type=jnp.float32)
                        + b_ref[...])


def kernel(x, idx_comp, edge_index, edge_weight, edge_attr,
           W_mlp0, b_mlp0, W_mlp2, b_mlp2, W_lin1, W_lin2, b_lin2, W_lin, b_lin):
    idx_comp = idx_comp.astype(jnp.int32)
    src = edge_index[0]
    dst = edge_index[1]
    eat = edge_attr.T
    ew3 = edge_weight.reshape(E // BE, 1, BE)
    w0t = W_mlp0.T
    b0r = b_mlp0.reshape(1, NF)
    w2t = W_mlp2.T
    b2r = b_mlp2.reshape(1, NF)

    def wf_half(half):
        boff = E_OFF[half] // BE
        return pl.pallas_call(
            _wf_body,
            grid=(E_SLICE[half] // BE,),
            in_specs=[
                pl.BlockSpec((NG, BE), lambda i, b=boff: (0, b + i)),
                pl.BlockSpec((1, 1, BE), lambda i, b=boff: (b + i, 0, 0)),
                pl.BlockSpec((NG, NF), lambda i: (0, 0)),
                pl.BlockSpec((1, NF), lambda i: (0, 0)),
                pl.BlockSpec((NF, NF), lambda i: (0, 0)),
                pl.BlockSpec((1, NF), lambda i: (0, 0)),
            ],
            out_specs=pl.BlockSpec((BE, NF), lambda i: (i, 0)),
            out_shape=jax.ShapeDtypeStruct((E_SLICE[half], NF), jnp.float32),
        )(eat, ew3, w0t, b0r, w2t, b2r)

    h = pl.pallas_call(
        _h_body,
        out_shape=jax.ShapeDtypeStruct((N, NF), jnp.float32),
    )(x, W_lin1.T)

    idx_pad = jnp.concatenate(
        [idx_comp, jnp.full((N_PAD - N,), NCOMP, jnp.int32)])

    mesh = plsc.VectorSubcoreMesh(core_axis_name="c", subcore_axis_name="s")
    sc_scratch = [
        pltpu.VMEM((max(E_SLICE) // NW,), jnp.int32),
        [pltpu.VMEM((CHUNK,), jnp.int32) for _ in range(NBUF)],
        [pltpu.VMEM((CHUNK, NF), jnp.float32) for _ in range(NBUF)],
        [pltpu.VMEM((CHUNK, NF), jnp.float32) for _ in range(NBUF)],
        pltpu.VMEM_SHARED((N_PAD, NF), jnp.float32),
        pltpu.SemaphoreType.DMA,
        [pltpu.SemaphoreType.DMA for _ in range(NBUF)],
        [pltpu.SemaphoreType.DMA for _ in range(NBUF)],
        [pltpu.SemaphoreType.DMA for _ in range(NBUF)],
    ]

    partials = []
    for half in range(NHALF):
        wf_h = wf_half(half)
        p = pl.kernel(
            functools.partial(_sc_body, half),
            out_type=jax.ShapeDtypeStruct((NC, N_PAD, NF), jnp.float32),
            mesh=mesh,
            scratch_types=sc_scratch,
        )(h, wf_h, src, dst)
        partials.append(p)

    out = pl.pallas_call(
        _final_body,
        grid=(N_PAD // BN,),
        in_specs=[
            pl.BlockSpec((NC, BN, NF), lambda i: (0, i, 0)),
            pl.BlockSpec((NC, BN, NF), lambda i: (0, i, 0)),
            pl.BlockSpec((1, 1, BN), lambda i: (i, 0, 0)),
            pl.BlockSpec((NF, NF), lambda i: (0, 0)),
            pl.BlockSpec((1, NF), lambda i: (0, 0)),
            pl.BlockSpec((NF, NF), lambda i: (0, 0)),
            pl.BlockSpec((1, NF), lambda i: (0, 0)),
        ],
        out_specs=pl.BlockSpec((NCOMP, HID), lambda i: (0, 0)),
        out_shape=jax.ShapeDtypeStruct((NCOMP, HID), jnp.float32),
        scratch_shapes=[
            pltpu.VMEM((NCOMP, NF), jnp.float32),
            pltpu.VMEM((NCOMP, 1), jnp.float32),
        ],
    )(partials[0], partials[1], idx_pad.reshape(N_PAD // BN, 1, BN),
      W_lin2.T, b_lin2.reshape(1, HID), W_lin.T, b_lin.reshape(1, HID))
    return out


# speedup vs baseline: 1.0327x; 1.0327x over previous
import functools

import jax
import jax.numpy as jnp
import numpy as np
from jax import lax
from jax.experimental import pallas as pl
from jax.experimental.pallas import tpu as pltpu
from jax.experimental.pallas import tpu_sc as plsc

N = 10000
E = 320000
HID = 128
NG = 50
NF = 128
NCOMP = 256
CUTOFF = 10.0
SHIFT = float(np.log(2.0))
LOG2E = float(np.log2(np.e))
LN2 = float(np.log(2.0))

NHALF = 2
EH = E // NHALF
NC = 2
NS = 16
NW = NC * NS
EDGES_PT = EH // NW
CHUNK = 40
NCHUNK = EDGES_PT // CHUNK
NBUF = 3
N_PAD = 10240
ROWS_PT = N_PAD // NS
ZR = 40
BE = 3200
NBLK = EH // BE
BN = 2048


def _wf_body(eat_ref, ew_ref, w0t_ref, b0_ref, w2t_ref, b2_ref, wf_ref):
    t = lax.dot_general(eat_ref[...], w0t_ref[...], (((0,), (0,)), ((), ())),
                        preferred_element_type=jnp.float32)
    t = t + b0_ref[...]
    t = jnp.log2(1.0 + jnp.exp2(t * LOG2E)) * LN2 - SHIFT
    wf = jnp.dot(t, w2t_ref[...], preferred_element_type=jnp.float32) + b2_ref[...]
    ew = ew_ref[0].reshape(BE, 1)
    c = CUTOFF / (1e-10 + ew * ew) - 1.0
    wf_ref[...] = wf * c


def _h_body(x_ref, w_ref, h_ref):
    h_ref[...] = jnp.dot(x_ref[...], w_ref[...], preferred_element_type=jnp.float32)


def _sc_body(half, h_hbm, wf_hbm, src_hbm, dst_hbm, out_hbm,
             srcall_v, dst_v, rows_v, wf_v, agg_sh,
             ssrc, sd, sg, sw):
    c = lax.axis_index("c")
    s = lax.axis_index("s")
    wid = s * NC + c
    base = half * EH + wid * EDGES_PT

    pltpu.async_copy(src_hbm.at[pl.ds(base, EDGES_PT)], srcall_v, ssrc)

    def zrow(r, cy):
        for j in range(NF // 16):
            rows_v[0][r, pl.ds(16 * j, 16)] = jnp.zeros((16,), jnp.float32)
        return cy

    lax.fori_loop(0, ZR, zrow, 0)
    for k in range(ROWS_PT // ZR):
        pltpu.sync_copy(rows_v[0], agg_sh.at[pl.ds(s * ROWS_PT + k * ZR, ZR)])
    plsc.subcore_barrier()
    pltpu.make_async_copy(src_hbm.at[pl.ds(base, EDGES_PT)], srcall_v, ssrc).wait()

    def issue(ci, b):
        off = base + ci * CHUNK
        pltpu.async_copy(dst_hbm.at[pl.ds(off, CHUNK)], dst_v[b], sd[b])
        pltpu.async_copy(h_hbm.at[srcall_v.at[pl.ds(ci * CHUNK, CHUNK)]],
                         rows_v[b], sg[b])
        pltpu.async_copy(wf_hbm.at[pl.ds(off - half * EH, CHUNK)], wf_v[b], sw[b])

    def wait_in(ci, b):
        off = base + ci * CHUNK
        pltpu.make_async_copy(h_hbm.at[srcall_v.at[pl.ds(ci * CHUNK, CHUNK)]],
                              rows_v[b], sg[b]).wait()
        pltpu.make_async_copy(wf_hbm.at[pl.ds(off - half * EH, CHUNK)],
                              wf_v[b], sw[b]).wait()
        pltpu.make_async_copy(dst_hbm.at[pl.ds(off, CHUNK)], dst_v[b], sd[b]).wait()

    def process(b):
        @plsc.parallel_loop(0, CHUNK, step=1, unroll=4)
        def _(r):
            for j in range(NF // 16):
                rows_v[b][r, pl.ds(16 * j, 16)] = (
                    rows_v[b][r, pl.ds(16 * j, 16)] * wf_v[b][r, pl.ds(16 * j, 16)])

        pltpu.sync_copy(rows_v[b], agg_sh.at[dst_v[b]], add=True)

    issue(0, 0)
    issue(1, 1)

    def step(i, carry):
        for par in range(NBUF):
            @pl.when(lax.rem(i, NBUF) == par)
            def _(par=par):
                @pl.when(i + 2 < NCHUNK)
                def _():
                    issue(i + 2, (par + 2) % NBUF)

                wait_in(i, par)
                process(par)

        return carry

    lax.fori_loop(0, NCHUNK, step, 0)
    plsc.subcore_barrier()
    for k in range(ROWS_PT // ZR):
        pltpu.sync_copy(agg_sh.at[pl.ds(s * ROWS_PT + k * ZR, ZR)],
                        out_hbm.at[c, pl.ds(s * ROWS_PT + k * ZR, ZR)])


def _final_body(p0_ref, p1_ref, idxc_ref, w2t_ref, b2_ref, wt_ref, b_ref, out_ref,
                acc_ref, cnt_ref):
    i = pl.program_id(0)
    agg = (p0_ref[0] + p0_ref[1]) + (p1_ref[0] + p1_ref[1])
    iota_c = lax.broadcasted_iota(jnp.int32, (NCOMP, BN), 0)
    sel = (idxc_ref[0] == iota_c).astype(jnp.float32)
    acc = jnp.dot(sel, agg, preferred_element_type=jnp.float32)
    cnt = jnp.sum(sel, axis=1, keepdims=True)

    @pl.when(i == 0)
    def _():
        acc_ref[...] = jnp.zeros_like(acc_ref)
        cnt_ref[...] = jnp.zeros_like(cnt_ref)

    acc_ref[...] += acc
    cnt_ref[...] += cnt

    @pl.when(i == pl.num_programs(0) - 1)
    def _():
        mean = acc_ref[...] / jnp.maximum(cnt_ref[...], 1.0)
        t = jnp.dot(mean, w2t_ref[...], preferred_element_type=jnp.float32)
        t = jax.nn.softplus(t + b2_ref[...]) - SHIFT
        out_ref[...] = (jnp.dot(t, wt_ref[...], preferred_element_type=jnp.float32)
                        + b_ref[...])


def kernel(x, idx_comp, edge_index, edge_weight, edge_attr,
           W_mlp0, b_mlp0, W_mlp2, b_mlp2, W_lin1, W_lin2, b_lin2, W_lin, b_lin):
    idx_comp = idx_comp.astype(jnp.int32)
    src = edge_index[0]
    dst = edge_index[1]
    eat = edge_attr.T
    ew3 = edge_weight.reshape(E // BE, 1, BE)
    w0t = W_mlp0.T
    b0r = b_mlp0.reshape(1, NF)
    w2t = W_mlp2.T
    b2r = b_mlp2.reshape(1, NF)

    def wf_half(half):
        return pl.pallas_call(
            _wf_body,
            grid=(NBLK,),
            in_specs=[
                pl.BlockSpec((NG, BE), lambda i, h=half: (0, h * NBLK + i)),
                pl.BlockSpec((1, 1, BE), lambda i, h=half: (h * NBLK + i, 0, 0)),
                pl.BlockSpec((NG, NF), lambda i: (0, 0)),
                pl.BlockSpec((1, NF), lambda i: (0, 0)),
                pl.BlockSpec((NF, NF), lambda i: (0, 0)),
                pl.BlockSpec((1, NF), lambda i: (0, 0)),
            ],
            out_specs=pl.BlockSpec((BE, NF), lambda i: (i, 0)),
            out_shape=jax.ShapeDtypeStruct((EH, NF), jnp.float32),
        )(eat, ew3, w0t, b0r, w2t, b2r)

    h = pl.pallas_call(
        _h_body,
        out_shape=jax.ShapeDtypeStruct((N, NF), jnp.float32),
    )(x, W_lin1.T)

    idx_pad = jnp.concatenate(
        [idx_comp, jnp.full((N_PAD - N,), NCOMP, jnp.int32)])

    mesh = plsc.VectorSubcoreMesh(core_axis_name="c", subcore_axis_name="s")
    sc_scratch = [
        pltpu.VMEM((EDGES_PT,), jnp.int32),
        [pltpu.VMEM((CHUNK,), jnp.int32) for _ in range(NBUF)],
        [pltpu.VMEM((CHUNK, NF), jnp.float32) for _ in range(NBUF)],
        [pltpu.VMEM((CHUNK, NF), jnp.float32) for _ in range(NBUF)],
        pltpu.VMEM_SHARED((N_PAD, NF), jnp.float32),
        pltpu.SemaphoreType.DMA,
        [pltpu.SemaphoreType.DMA for _ in range(NBUF)],
        [pltpu.SemaphoreType.DMA for _ in range(NBUF)],
        [pltpu.SemaphoreType.DMA for _ in range(NBUF)],
    ]

    partials = []
    for half in range(NHALF):
        wf_h = wf_half(half)
        p = pl.kernel(
            functools.partial(_sc_body, half),
            out_type=jax.ShapeDtypeStruct((NC, N_PAD, NF), jnp.float32),
            mesh=mesh,
            scratch_types=sc_scratch,
        )(h, wf_h, src, dst)
        partials.append(p)

    out = pl.pallas_call(
        _final_body,
        grid=(N_PAD // BN,),
        in_specs=[
            pl.BlockSpec((NC, BN, NF), lambda i: (0, i, 0)),
            pl.BlockSpec((NC, BN, NF), lambda i: (0, i, 0)),
            pl.BlockSpec((1, 1, BN), lambda i: (i, 0, 0)),
            pl.BlockSpec((NF, NF), lambda i: (0, 0)),
            pl.BlockSpec((1, NF), lambda i: (0, 0)),
            pl.BlockSpec((NF, NF), lambda i: (0, 0)),
            pl.BlockSpec((1, NF), lambda i: (0, 0)),
        ],
        out_specs=pl.BlockSpec((NCOMP, HID), lambda i: (0, 0)),
        out_shape=jax.ShapeDtypeStruct((NCOMP, HID), jnp.float32),
        scratch_shapes=[
            pltpu.VMEM((NCOMP, NF), jnp.float32),
            pltpu.VMEM((NCOMP, 1), jnp.float32),
        ],
    )(partials[0], partials[1], idx_pad.reshape(N_PAD // BN, 1, BN),
      W_lin2.T, b_lin2.reshape(1, HID), W_lin.T, b_lin.reshape(1, HID))
    return out
